# baseline (speedup 1.0000x reference)
import jax
import jax.numpy as jnp
from jax.experimental import pallas as pl
from jax.experimental.pallas import tpu as pltpu

_L_BLOCK = 512


def _add_body(x_ref, t_ref, o_ref):
    o_ref[...] = x_ref[...] + t_ref[...][None, :, :]


def kernel(x, pos_table):
    B, L, D = x.shape
    lb = min(_L_BLOCK, L)
    return pl.pallas_call(
        _add_body,
        grid=(L // lb,),
        in_specs=[
            pl.BlockSpec((B, lb, D), lambda i: (0, i, 0)),
            pl.BlockSpec((lb, D), lambda i: (i, 0)),
        ],
        out_specs=pl.BlockSpec((B, lb, D), lambda i: (0, i, 0)),
        out_shape=jax.ShapeDtypeStruct((B, L, D), x.dtype),
        compiler_params=pltpu.CompilerParams(
            dimension_semantics=("parallel",),
        ),
    )(x, pos_table[:L])
